# baseline (speedup 1.0000x reference)
import functools

import jax
import jax.numpy as jnp
from jax import lax
from jax.experimental import pallas as pl
from jax.experimental.pallas import tpu as pltpu
from jax.experimental.pallas import tpu_sc as plsc

_N = 10000
_E = 320000
_F_IN = 128
_DIM = 32
_OUT = 128
_G = 64

_NC = 2
_NS = 16
_NW = _NC * _NS
_CHUNK = 256
_NB = 8
_PF = 4
_NCH = 80
_TOT_CH = _NCH * _NS
_E_PAD = _TOT_CH * _CHUNK
_N_PAD = 10240
_RPT = _N_PAD // _NS


@functools.cache
def _make_sc_segsum():
    mesh = plsc.VectorSubcoreMesh(
        core_axis_name="c", subcore_axis_name="s",
        num_cores=1, num_subcores=_NS)

    @functools.partial(
        pl.kernel,
        out_type=jax.ShapeDtypeStruct((_N_PAD, _DIM), jnp.float32),
        mesh=mesh,
        scratch_types=[
            pltpu.VMEM((_NCH, _CHUNK), jnp.int32),
            pltpu.VMEM((_NCH, _CHUNK), jnp.int32),
            [pltpu.VMEM((_CHUNK, _DIM), jnp.float32) for _ in range(_NB)],
            pltpu.VMEM_SHARED((_N_PAD, _DIM), jnp.float32),
            [pltpu.SemaphoreType.DMA for _ in range(_NB)],
            [pltpu.SemaphoreType.DMA for _ in range(_NB)],
        ],
        compiler_params=pltpu.CompilerParams(use_tc_tiling_on_sc=False),
    )
    def seg(u_hbm, srcp_hbm, dstp_hbm, zeros_hbm, out_hbm,
            src_v, dst_v, rows_v, acc_sh, gsems, ssems):
        sid = lax.axis_index("s")

        pltpu.sync_copy(srcp_hbm.at[pl.ds(sid * _NCH, _NCH)], src_v)
        pltpu.sync_copy(dstp_hbm.at[pl.ds(sid * _NCH, _NCH)], dst_v)

        pltpu.sync_copy(zeros_hbm.at[pl.ds(sid * _RPT, _RPT)],
                        acc_sh.at[pl.ds(sid * _RPT, _RPT)])
        plsc.subcore_barrier()

        def gather(j, b):
            pltpu.async_copy(u_hbm.at[src_v.at[j]], rows_v[b], gsems[b])

        def wait_gather(j, b):
            pltpu.make_async_copy(u_hbm.at[src_v.at[j]], rows_v[b],
                                  gsems[b]).wait()

        def scatter(j, b):
            pltpu.async_copy(rows_v[b], acc_sh.at[dst_v.at[j]], ssems[b],
                             add=True)

        def wait_scatter(j, b):
            pltpu.make_async_copy(rows_v[b], acc_sh.at[dst_v.at[j]],
                                  ssems[b]).wait()

        for b in range(_PF):
            gather(b, b)

        def body(g, carry):
            for b in range(_NB):
                j = g * _NB + b
                wait_gather(j, b)
                scatter(j, b)
                pre = j + _PF
                pb = (b + _PF) % _NB

                @pl.when(jnp.logical_and(pre < _NCH, pre >= _NB))
                def _():
                    wait_scatter(pre - _NB, pb)

                @pl.when(pre < _NCH)
                def _():
                    gather(pre, pb)
            return carry

        lax.fori_loop(0, _NCH // _NB, body, 0)
        for k in range(_NB):
            wait_scatter(_NCH - _NB + k, k)
        plsc.subcore_barrier()
        pltpu.sync_copy(acc_sh.at[pl.ds(sid * _RPT, _RPT)],
                        out_hbm.at[pl.ds(sid * _RPT, _RPT)])

    return seg


def _dot(a, b):
    return jnp.dot(a, b, preferred_element_type=jnp.float32)


def _dense_block(u, agg, b1, w2, b2, gamma, beta):
    z = jnp.maximum(u + agg + b1, 0.0)
    z = _dot(z, w2) + b2
    mean = jnp.mean(z, axis=0, keepdims=True)
    var = jnp.mean(jnp.square(z - mean), axis=0, keepdims=True)
    z = gamma * (z - mean) / jnp.sqrt(var + 1e-5) + beta
    return jnp.maximum(z, 0.0)


def _tc_proj(x, w):
    def body(x_ref, w_ref, o_ref):
        o_ref[...] = _dot(x_ref[...], w_ref[...])

    return pl.pallas_call(
        body, out_shape=jax.ShapeDtypeStruct((_N, _DIM), jnp.float32))(x, w)


def _tc_layer(u, aggp, b1, w2, b2, gamma, beta, w1n):
    def body(u_ref, agg_ref, b1_ref, w2_ref, b2_ref, g_ref, be_ref,
             w1n_ref, o_ref):
        agg = agg_ref[:_N, :]
        h = _dense_block(u_ref[...], agg, b1_ref[...], w2_ref[...],
                         b2_ref[...], g_ref[...], be_ref[...])
        o_ref[...] = _dot(h, w1n_ref[...])

    return pl.pallas_call(
        body, out_shape=jax.ShapeDtypeStruct((_N, _DIM), jnp.float32))(
            u, aggp, b1, w2, b2, gamma, beta, w1n)


def _tc_final(u, aggp, b1, w2, b2, gamma, beta, gid2d, wfc, bfc):
    def body(u_ref, agg_ref, b1_ref, w2_ref, b2_ref, g_ref, be_ref,
             gid_ref, wfc_ref, bfc_ref, o_ref):
        agg = agg_ref[:_N, :]
        h = _dense_block(u_ref[...], agg, b1_ref[...], w2_ref[...],
                         b2_ref[...], g_ref[...], be_ref[...])
        gid = jnp.broadcast_to(gid_ref[...], (_G, _N))
        rows = lax.broadcasted_iota(jnp.int32, (_G, _N), 0)
        onehot = (gid == rows).astype(jnp.float32)
        pooled = _dot(onehot, h)
        o_ref[...] = jnp.maximum(_dot(pooled, wfc_ref[...]) + bfc_ref[...], 0.0)

    return pl.pallas_call(
        body, out_shape=jax.ShapeDtypeStruct((_G, _OUT), jnp.float32))(
            u, aggp, b1, w2, b2, gamma, beta, gid2d, wfc, bfc)


def kernel(x, edge_index, graph_id, params):
    src = edge_index[0]
    dst = edge_index[1]
    pad = _E_PAD - _E
    srcp = jnp.concatenate(
        [src, jnp.zeros((pad,), jnp.int32)]).reshape(_TOT_CH, _CHUNK)
    dstp = jnp.concatenate(
        [dst, jnp.full((pad,), _N_PAD - 1, jnp.int32)]).reshape(_TOT_CH, _CHUNK)
    zeros = jnp.zeros((_N_PAD, _DIM), jnp.float32)
    gid2d = graph_id.reshape(1, _N)

    sc_segsum = _make_sc_segsum()
    u = _tc_proj(x, params["layer1"]["W1"])
    out = None
    for i in range(1, 6):
        p = params[f"layer{i}"]
        aggp = sc_segsum(u, srcp, dstp, zeros)
        b1 = p["b1"].reshape(1, _DIM)
        b2 = p["b2"].reshape(1, _DIM)
        gamma = p["gamma"].reshape(1, _DIM)
        beta = p["beta"].reshape(1, _DIM)
        if i < 5:
            w1n = params[f"layer{i + 1}"]["W1"]
            u = _tc_layer(u, aggp, b1, p["W2"], b2, gamma, beta, w1n)
        else:
            out = _tc_final(u, aggp, b1, p["W2"], b2, gamma, beta, gid2d,
                            params["fc"]["W"], params["fc"]["b"].reshape(1, _OUT))
    return out


# speedup vs baseline: 1.0427x; 1.0427x over previous
import functools

import jax
import jax.numpy as jnp
from jax import lax
from jax.experimental import pallas as pl
from jax.experimental.pallas import tpu as pltpu
from jax.experimental.pallas import tpu_sc as plsc

_N = 10000
_E = 320000
_F_IN = 128
_DIM = 32
_OUT = 128
_G = 64

_NC = 2
_NS = 16
_NW = _NC * _NS
_CHUNK = 256
_NB = 8
_PF = 4
_NCH = 80
_TOT_CH = _NCH * _NS
_E_PAD = _TOT_CH * _CHUNK
_N_PAD = 10240
_RPT = _N_PAD // _NS


@functools.cache
def _make_sc_segsum():
    mesh = plsc.VectorSubcoreMesh(
        core_axis_name="c", subcore_axis_name="s",
        num_cores=1, num_subcores=_NS)

    @functools.partial(
        pl.kernel,
        out_type=jax.ShapeDtypeStruct((_N_PAD, _DIM), jnp.float32),
        mesh=mesh,
        scratch_types=[
            pltpu.VMEM((_NCH, _CHUNK), jnp.int32),
            pltpu.VMEM((_NCH, _CHUNK), jnp.int32),
            [pltpu.VMEM((_CHUNK, _DIM), jnp.float32) for _ in range(_NB)],
            pltpu.VMEM_SHARED((_N_PAD, _DIM), jnp.float32),
            [pltpu.SemaphoreType.DMA for _ in range(_NB)],
            [pltpu.SemaphoreType.DMA for _ in range(_NB)],
        ],
        compiler_params=pltpu.CompilerParams(use_tc_tiling_on_sc=False),
    )
    def seg(u_hbm, srcp_hbm, dstp_hbm, zeros_hbm, out_hbm,
            src_v, dst_v, rows_v, acc_sh, gsems, ssems):
        sid = lax.axis_index("s")

        pltpu.sync_copy(srcp_hbm.at[pl.ds(sid * _NCH, _NCH)], src_v)
        pltpu.sync_copy(dstp_hbm.at[pl.ds(sid * _NCH, _NCH)], dst_v)

        pltpu.sync_copy(zeros_hbm.at[pl.ds(sid * _RPT, _RPT)],
                        acc_sh.at[pl.ds(sid * _RPT, _RPT)])
        plsc.subcore_barrier()

        def gather(j, b):
            pltpu.async_copy(u_hbm.at[src_v.at[j]], rows_v[b], gsems[b])

        def wait_gather(j, b):
            pltpu.make_async_copy(u_hbm.at[src_v.at[j]], rows_v[b],
                                  gsems[b]).wait()

        def scatter(j, b):
            pltpu.async_copy(rows_v[b], acc_sh.at[dst_v.at[j]], ssems[b],
                             add=True)

        def wait_scatter(j, b):
            pltpu.make_async_copy(rows_v[b], acc_sh.at[dst_v.at[j]],
                                  ssems[b]).wait()

        for b in range(_PF):
            gather(b, b)

        def body(g, carry):
            for b in range(_NB):
                j = g * _NB + b
                wait_gather(j, b)
                pre = j + _PF
                pb = (b + _PF) % _NB

                @pl.when(pre < _NCH)
                def _():
                    gather(pre, pb)
            return carry

        lax.fori_loop(0, _NCH // _NB, body, 0)
        plsc.subcore_barrier()
        pltpu.sync_copy(acc_sh.at[pl.ds(sid * _RPT, _RPT)],
                        out_hbm.at[pl.ds(sid * _RPT, _RPT)])

    return seg


def _dot(a, b):
    return jnp.dot(a, b, preferred_element_type=jnp.float32)


def _dense_block(u, agg, b1, w2, b2, gamma, beta):
    z = jnp.maximum(u + agg + b1, 0.0)
    z = _dot(z, w2) + b2
    mean = jnp.mean(z, axis=0, keepdims=True)
    var = jnp.mean(jnp.square(z - mean), axis=0, keepdims=True)
    z = gamma * (z - mean) / jnp.sqrt(var + 1e-5) + beta
    return jnp.maximum(z, 0.0)


def _tc_proj(x, w):
    def body(x_ref, w_ref, o_ref):
        o_ref[...] = _dot(x_ref[...], w_ref[...])

    return pl.pallas_call(
        body, out_shape=jax.ShapeDtypeStruct((_N, _DIM), jnp.float32))(x, w)


def _tc_layer(u, aggp, b1, w2, b2, gamma, beta, w1n):
    def body(u_ref, agg_ref, b1_ref, w2_ref, b2_ref, g_ref, be_ref,
             w1n_ref, o_ref):
        agg = agg_ref[:_N, :]
        h = _dense_block(u_ref[...], agg, b1_ref[...], w2_ref[...],
                         b2_ref[...], g_ref[...], be_ref[...])
        o_ref[...] = _dot(h, w1n_ref[...])

    return pl.pallas_call(
        body, out_shape=jax.ShapeDtypeStruct((_N, _DIM), jnp.float32))(
            u, aggp, b1, w2, b2, gamma, beta, w1n)


def _tc_final(u, aggp, b1, w2, b2, gamma, beta, gid2d, wfc, bfc):
    def body(u_ref, agg_ref, b1_ref, w2_ref, b2_ref, g_ref, be_ref,
             gid_ref, wfc_ref, bfc_ref, o_ref):
        agg = agg_ref[:_N, :]
        h = _dense_block(u_ref[...], agg, b1_ref[...], w2_ref[...],
                         b2_ref[...], g_ref[...], be_ref[...])
        gid = jnp.broadcast_to(gid_ref[...], (_G, _N))
        rows = lax.broadcasted_iota(jnp.int32, (_G, _N), 0)
        onehot = (gid == rows).astype(jnp.float32)
        pooled = _dot(onehot, h)
        o_ref[...] = jnp.maximum(_dot(pooled, wfc_ref[...]) + bfc_ref[...], 0.0)

    return pl.pallas_call(
        body, out_shape=jax.ShapeDtypeStruct((_G, _OUT), jnp.float32))(
            u, aggp, b1, w2, b2, gamma, beta, gid2d, wfc, bfc)


def kernel(x, edge_index, graph_id, params):
    src = edge_index[0]
    dst = edge_index[1]
    pad = _E_PAD - _E
    srcp = jnp.concatenate(
        [src, jnp.zeros((pad,), jnp.int32)]).reshape(_TOT_CH, _CHUNK)
    dstp = jnp.concatenate(
        [dst, jnp.full((pad,), _N_PAD - 1, jnp.int32)]).reshape(_TOT_CH, _CHUNK)
    zeros = jnp.zeros((_N_PAD, _DIM), jnp.float32)
    gid2d = graph_id.reshape(1, _N)

    sc_segsum = _make_sc_segsum()
    u = _tc_proj(x, params["layer1"]["W1"])
    out = None
    for i in range(1, 6):
        p = params[f"layer{i}"]
        aggp = sc_segsum(u, srcp, dstp, zeros)
        b1 = p["b1"].reshape(1, _DIM)
        b2 = p["b2"].reshape(1, _DIM)
        gamma = p["gamma"].reshape(1, _DIM)
        beta = p["beta"].reshape(1, _DIM)
        if i < 5:
            w1n = params[f"layer{i + 1}"]["W1"]
            u = _tc_layer(u, aggp, b1, p["W2"], b2, gamma, beta, w1n)
        else:
            out = _tc_final(u, aggp, b1, p["W2"], b2, gamma, beta, gid2d,
                            params["fc"]["W"], params["fc"]["b"].reshape(1, _OUT))
    return out


# speedup vs baseline: 2.1167x; 2.0299x over previous
import functools

import jax
import jax.numpy as jnp
from jax import lax
from jax.experimental import pallas as pl
from jax.experimental.pallas import tpu as pltpu
from jax.experimental.pallas import tpu_sc as plsc

_N = 10000
_E = 320000
_F_IN = 128
_DIM = 32
_OUT = 128
_G = 64

_NC = 2
_NS = 16
_NW = _NC * _NS
_CHUNK = 256
_NB = 8
_PF = 4
_NCH = 40
_TOT_CH = _NCH * _NW
_E_PAD = _TOT_CH * _CHUNK
_N_PAD = 10240
_RPT = _N_PAD // _NS


@functools.cache
def _make_sc_segsum():
    mesh = plsc.VectorSubcoreMesh(
        core_axis_name="c", subcore_axis_name="s",
        num_cores=_NC, num_subcores=_NS)

    @functools.partial(
        pl.kernel,
        out_type=jax.ShapeDtypeStruct((_NC, _N_PAD, _DIM), jnp.float32),
        mesh=mesh,
        scratch_types=[
            pltpu.VMEM((_NCH, _CHUNK), jnp.int32),
            pltpu.VMEM((_NCH, _CHUNK), jnp.int32),
            [pltpu.VMEM((_CHUNK, _DIM), jnp.float32) for _ in range(_NB)],
            pltpu.VMEM_SHARED((_N, _DIM), jnp.float32),
            pltpu.VMEM_SHARED((_N_PAD, _DIM), jnp.float32),
            [pltpu.SemaphoreType.DMA for _ in range(_NB)],
            [pltpu.SemaphoreType.DMA for _ in range(_NB)],
        ],
        compiler_params=pltpu.CompilerParams(use_tc_tiling_on_sc=False),
    )
    def seg(u_hbm, srcp_hbm, dstp_hbm, zeros_hbm, out_hbm,
            src_v, dst_v, rows_v, u_sh, acc_sh, gsems, ssems):
        cid = lax.axis_index("c")
        sid = lax.axis_index("s")
        wid = cid * _NS + sid

        pltpu.sync_copy(srcp_hbm.at[pl.ds(wid * _NCH, _NCH)], src_v)
        pltpu.sync_copy(dstp_hbm.at[pl.ds(wid * _NCH, _NCH)], dst_v)

        @pl.when(sid < _NS - 1)
        def _():
            pltpu.sync_copy(u_hbm.at[pl.ds(sid * _RPT, _RPT)],
                            u_sh.at[pl.ds(sid * _RPT, _RPT)])

        @pl.when(sid == _NS - 1)
        def _():
            pltpu.sync_copy(u_hbm.at[pl.ds((_NS - 1) * _RPT, _N - (_NS - 1) * _RPT)],
                            u_sh.at[pl.ds((_NS - 1) * _RPT, _N - (_NS - 1) * _RPT)])

        pltpu.sync_copy(zeros_hbm.at[pl.ds(sid * _RPT, _RPT)],
                        acc_sh.at[pl.ds(sid * _RPT, _RPT)])
        plsc.subcore_barrier()

        def gather(j, b):
            pltpu.async_copy(u_sh.at[src_v.at[j]], rows_v[b], gsems[b])

        def wait_gather(j, b):
            pltpu.make_async_copy(u_sh.at[src_v.at[j]], rows_v[b],
                                  gsems[b]).wait()

        def scatter(j, b):
            pltpu.async_copy(rows_v[b], acc_sh.at[dst_v.at[j]], ssems[b],
                             add=True)

        def wait_scatter(j, b):
            pltpu.make_async_copy(rows_v[b], acc_sh.at[dst_v.at[j]],
                                  ssems[b]).wait()

        for b in range(_PF):
            gather(b, b)

        def body(g, carry):
            for b in range(_NB):
                j = g * _NB + b
                wait_gather(j, b)
                scatter(j, b)
                pre = j + _PF
                pb = (b + _PF) % _NB

                @pl.when(jnp.logical_and(pre < _NCH, pre >= _NB))
                def _():
                    wait_scatter(pre - _NB, pb)

                @pl.when(pre < _NCH)
                def _():
                    gather(pre, pb)
            return carry

        lax.fori_loop(0, _NCH // _NB, body, 0)
        for k in range(_NB):
            wait_scatter(_NCH - _NB + k, k)
        plsc.subcore_barrier()
        pltpu.sync_copy(acc_sh.at[pl.ds(sid * _RPT, _RPT)],
                        out_hbm.at[cid, pl.ds(sid * _RPT, _RPT)])

    return seg


def _dot(a, b):
    return jnp.dot(a, b, preferred_element_type=jnp.float32)


def _dense_block(u, agg, b1, w2, b2, gamma, beta):
    z = jnp.maximum(u + agg + b1, 0.0)
    z = _dot(z, w2) + b2
    mean = jnp.mean(z, axis=0, keepdims=True)
    var = jnp.mean(jnp.square(z - mean), axis=0, keepdims=True)
    z = gamma * (z - mean) / jnp.sqrt(var + 1e-5) + beta
    return jnp.maximum(z, 0.0)


def _tc_proj(x, w):
    def body(x_ref, w_ref, o_ref):
        o_ref[...] = _dot(x_ref[...], w_ref[...])

    return pl.pallas_call(
        body, out_shape=jax.ShapeDtypeStruct((_N, _DIM), jnp.float32))(x, w)


def _tc_layer(u, aggp, b1, w2, b2, gamma, beta, w1n):
    def body(u_ref, agg_ref, b1_ref, w2_ref, b2_ref, g_ref, be_ref,
             w1n_ref, o_ref):
        agg = agg_ref[0, :_N, :] + agg_ref[1, :_N, :]
        h = _dense_block(u_ref[...], agg, b1_ref[...], w2_ref[...],
                         b2_ref[...], g_ref[...], be_ref[...])
        o_ref[...] = _dot(h, w1n_ref[...])

    return pl.pallas_call(
        body, out_shape=jax.ShapeDtypeStruct((_N, _DIM), jnp.float32))(
            u, aggp, b1, w2, b2, gamma, beta, w1n)


def _tc_final(u, aggp, b1, w2, b2, gamma, beta, gid2d, wfc, bfc):
    def body(u_ref, agg_ref, b1_ref, w2_ref, b2_ref, g_ref, be_ref,
             gid_ref, wfc_ref, bfc_ref, o_ref):
        agg = agg_ref[0, :_N, :] + agg_ref[1, :_N, :]
        h = _dense_block(u_ref[...], agg, b1_ref[...], w2_ref[...],
                         b2_ref[...], g_ref[...], be_ref[...])
        gid = jnp.broadcast_to(gid_ref[...], (_G, _N))
        rows = lax.broadcasted_iota(jnp.int32, (_G, _N), 0)
        onehot = (gid == rows).astype(jnp.float32)
        pooled = _dot(onehot, h)
        o_ref[...] = jnp.maximum(_dot(pooled, wfc_ref[...]) + bfc_ref[...], 0.0)

    return pl.pallas_call(
        body, out_shape=jax.ShapeDtypeStruct((_G, _OUT), jnp.float32))(
            u, aggp, b1, w2, b2, gamma, beta, gid2d, wfc, bfc)


def kernel(x, edge_index, graph_id, params):
    src = edge_index[0]
    dst = edge_index[1]
    pad = _E_PAD - _E
    srcp = jnp.concatenate(
        [src, jnp.zeros((pad,), jnp.int32)]).reshape(_TOT_CH, _CHUNK)
    dstp = jnp.concatenate(
        [dst, jnp.full((pad,), _N_PAD - 1, jnp.int32)]).reshape(_TOT_CH, _CHUNK)
    zeros = jnp.zeros((_N_PAD, _DIM), jnp.float32)
    gid2d = graph_id.reshape(1, _N)

    sc_segsum = _make_sc_segsum()
    u = _tc_proj(x, params["layer1"]["W1"])
    out = None
    for i in range(1, 6):
        p = params[f"layer{i}"]
        aggp = sc_segsum(u, srcp, dstp, zeros)
        b1 = p["b1"].reshape(1, _DIM)
        b2 = p["b2"].reshape(1, _DIM)
        gamma = p["gamma"].reshape(1, _DIM)
        beta = p["beta"].reshape(1, _DIM)
        if i < 5:
            w1n = params[f"layer{i + 1}"]["W1"]
            u = _tc_layer(u, aggp, b1, p["W2"], b2, gamma, beta, w1n)
        else:
            out = _tc_final(u, aggp, b1, p["W2"], b2, gamma, beta, gid2d,
                            params["fc"]["W"], params["fc"]["b"].reshape(1, _OUT))
    return out


# speedup vs baseline: 2.8311x; 1.3375x over previous
import functools

import jax
import jax.numpy as jnp
from jax import lax
from jax.experimental import pallas as pl
from jax.experimental.pallas import tpu as pltpu
from jax.experimental.pallas import tpu_sc as plsc

_N = 10000
_E = 320000
_F_IN = 128
_DIM = 32
_OUT = 128
_G = 64

_NC = 2
_NS = 16
_NW = _NC * _NS
_CHUNK = 256
_NB = 8
_PF = 4
_NCH = 40
_TOT_CH = _NCH * _NW
_E_PAD = _TOT_CH * _CHUNK
_N_PAD = 10240
_RPT = _N_PAD // _NS
_NR = _N * _DIM // 128
_NRP = _N_PAD * _DIM // 128


@functools.cache
def _make_sc_segsum():
    mesh = plsc.VectorSubcoreMesh(
        core_axis_name="c", subcore_axis_name="s",
        num_cores=_NC, num_subcores=_NS)

    @functools.partial(
        pl.kernel,
        out_type=jax.ShapeDtypeStruct((_NC, _N_PAD, _DIM), jnp.float32),
        mesh=mesh,
        scratch_types=[
            pltpu.VMEM((_NCH, _CHUNK), jnp.int32),
            pltpu.VMEM((_NCH, _CHUNK), jnp.int32),
            [pltpu.VMEM((_CHUNK, _DIM), jnp.float32) for _ in range(_NB)],
            pltpu.VMEM_SHARED((_N, _DIM), jnp.float32),
            pltpu.VMEM_SHARED((_N_PAD, _DIM), jnp.float32),
            [pltpu.SemaphoreType.DMA for _ in range(_NB)],
            [pltpu.SemaphoreType.DMA for _ in range(_NB)],
        ],
        compiler_params=pltpu.CompilerParams(use_tc_tiling_on_sc=False),
    )
    def seg(u_hbm, srcp_hbm, dstp_hbm, zeros_hbm, out_hbm,
            src_v, dst_v, rows_v, u_sh, acc_sh, gsems, ssems):
        cid = lax.axis_index("c")
        sid = lax.axis_index("s")
        wid = cid * _NS + sid

        pltpu.sync_copy(srcp_hbm.at[pl.ds(wid * _NCH, _NCH)], src_v)
        pltpu.sync_copy(dstp_hbm.at[pl.ds(wid * _NCH, _NCH)], dst_v)

        @pl.when(sid < _NS - 1)
        def _():
            pltpu.sync_copy(u_hbm.at[pl.ds(sid * _RPT, _RPT)],
                            u_sh.at[pl.ds(sid * _RPT, _RPT)])

        @pl.when(sid == _NS - 1)
        def _():
            pltpu.sync_copy(
                u_hbm.at[pl.ds((_NS - 1) * _RPT, _N - (_NS - 1) * _RPT)],
                u_sh.at[pl.ds((_NS - 1) * _RPT, _N - (_NS - 1) * _RPT)])

        pltpu.sync_copy(zeros_hbm.at[pl.ds(sid * _RPT, _RPT)],
                        acc_sh.at[pl.ds(sid * _RPT, _RPT)])
        plsc.subcore_barrier()

        def gather(j, b):
            pltpu.async_copy(u_sh.at[src_v.at[j]], rows_v[b], gsems[b])

        def wait_gather(j, b):
            pltpu.make_async_copy(u_sh.at[src_v.at[j]], rows_v[b],
                                  gsems[b]).wait()

        def scatter(j, b):
            pltpu.async_copy(rows_v[b], acc_sh.at[dst_v.at[j]], ssems[b],
                             add=True)

        def wait_scatter(j, b):
            pltpu.make_async_copy(rows_v[b], acc_sh.at[dst_v.at[j]],
                                  ssems[b]).wait()

        for b in range(_PF):
            gather(b, b)

        def body(g, carry):
            for b in range(_NB):
                j = g * _NB + b
                wait_gather(j, b)
                scatter(j, b)
                pre = j + _PF
                pb = (b + _PF) % _NB

                @pl.when(jnp.logical_and(pre < _NCH, pre >= _NB))
                def _():
                    wait_scatter(pre - _NB, pb)

                @pl.when(pre < _NCH)
                def _():
                    gather(pre, pb)
            return carry

        lax.fori_loop(0, _NCH // _NB, body, 0)
        for k in range(_NB):
            wait_scatter(_NCH - _NB + k, k)
        plsc.subcore_barrier()
        pltpu.sync_copy(acc_sh.at[pl.ds(sid * _RPT, _RPT)],
                        out_hbm.at[cid, pl.ds(sid * _RPT, _RPT)])

    return seg


def _dot(a, b):
    return jnp.dot(a, b, preferred_element_type=jnp.float32)


def _combine4(s):
    c = (s[:, 0 * _DIM:1 * _DIM] + s[:, 1 * _DIM:2 * _DIM]
         + s[:, 2 * _DIM:3 * _DIM] + s[:, 3 * _DIM:4 * _DIM])
    return jnp.concatenate([c, c, c, c], axis=1)


def _dense_block(u, agg, b1, w2blk, b2, gamma, beta):
    z = jnp.maximum(u + agg + b1, 0.0)
    z = _dot(z, w2blk) + b2
    mu = _combine4(jnp.sum(z, axis=0, keepdims=True)) * (1.0 / _N)
    d = z - mu
    var = _combine4(jnp.sum(d * d, axis=0, keepdims=True)) * (1.0 / _N)
    z = gamma * d / jnp.sqrt(var + 1e-5) + beta
    return jnp.maximum(z, 0.0)


def _tc_proj(x, w):
    def body(x_ref, w_ref, o_ref):
        o_ref[...] = _dot(x_ref[...], w_ref[...])

    return pl.pallas_call(
        body, out_shape=jax.ShapeDtypeStruct((_N, _DIM), jnp.float32))(x, w)


def _tc_layer(u, aggp, b1, w2blk, b2, gamma, beta, w1nblk):
    def body(u_ref, agg_ref, b1_ref, w2_ref, b2_ref, g_ref, be_ref,
             w1n_ref, o_ref):
        agg = agg_ref[0, :_NR, :] + agg_ref[1, :_NR, :]
        h = _dense_block(u_ref[...], agg, b1_ref[...], w2_ref[...],
                         b2_ref[...], g_ref[...], be_ref[...])
        o_ref[...] = _dot(h, w1n_ref[...])

    return pl.pallas_call(
        body, out_shape=jax.ShapeDtypeStruct((_NR, 128), jnp.float32))(
            u, aggp, b1, w2blk, b2, gamma, beta, w1nblk)


def _tc_final(u, aggp, b1, w2blk, b2, gamma, beta, gidt, wfc, bfc):
    def body(u_ref, agg_ref, b1_ref, w2_ref, b2_ref, g_ref, be_ref,
             gid_ref, wfc_ref, bfc_ref, o_ref):
        agg = agg_ref[0, :_NR, :] + agg_ref[1, :_NR, :]
        h = _dense_block(u_ref[...], agg, b1_ref[...], w2_ref[...],
                         b2_ref[...], g_ref[...], be_ref[...])
        rows = lax.broadcasted_iota(jnp.int32, (_G, _NR), 0)
        pooled = jnp.zeros((_G, _DIM), jnp.float32)
        for k in range(4):
            onek = (jnp.broadcast_to(gid_ref[k:k + 1, :], (_G, _NR))
                    == rows).astype(jnp.float32)
            pooled = pooled + _dot(onek, h[:, _DIM * k:_DIM * (k + 1)])
        o_ref[...] = jnp.maximum(_dot(pooled, wfc_ref[...]) + bfc_ref[...],
                                 0.0)

    return pl.pallas_call(
        body, out_shape=jax.ShapeDtypeStruct((_G, _OUT), jnp.float32))(
            u, aggp, b1, w2blk, b2, gamma, beta, gidt, wfc, bfc)


def _blockdiag(w):
    return jnp.kron(jnp.eye(4, dtype=jnp.float32), w)


def _tile4(v):
    return jnp.tile(v.reshape(1, _DIM), (1, 4))


def kernel(x, edge_index, graph_id, params):
    src = edge_index[0]
    dst = edge_index[1]
    pad = _E_PAD - _E
    srcp = jnp.concatenate(
        [src, jnp.zeros((pad,), jnp.int32)]).reshape(_TOT_CH, _CHUNK)
    dstp = jnp.concatenate(
        [dst, jnp.full((pad,), _N_PAD - 1, jnp.int32)]).reshape(_TOT_CH, _CHUNK)
    zeros = jnp.zeros((_N_PAD, _DIM), jnp.float32)
    gidt = graph_id.reshape(_NR, 4).T

    sc_segsum = _make_sc_segsum()
    u = _tc_proj(x, params["layer1"]["W1"]).reshape(_NR, 128)
    out = None
    for i in range(1, 6):
        p = params[f"layer{i}"]
        aggp = sc_segsum(u.reshape(_N, _DIM), srcp, dstp, zeros)
        agg128 = aggp.reshape(_NC, _NRP, 128)
        b1 = _tile4(p["b1"])
        b2 = _tile4(p["b2"])
        gamma = _tile4(p["gamma"])
        beta = _tile4(p["beta"])
        w2blk = _blockdiag(p["W2"])
        if i < 5:
            w1nblk = _blockdiag(params[f"layer{i + 1}"]["W1"])
            u = _tc_layer(u, agg128, b1, w2blk, b2, gamma, beta, w1nblk)
        else:
            out = _tc_final(u, agg128, b1, w2blk, b2, gamma, beta, gidt,
                            params["fc"]["W"], params["fc"]["b"].reshape(1, _OUT))
    return out


# speedup vs baseline: 2.8884x; 1.0203x over previous
import functools

import jax
import jax.numpy as jnp
from jax import lax
from jax.experimental import pallas as pl
from jax.experimental.pallas import tpu as pltpu
from jax.experimental.pallas import tpu_sc as plsc

_N = 10000
_E = 320000
_F_IN = 128
_DIM = 32
_OUT = 128
_G = 64

_NC = 2
_NS = 16
_NW = _NC * _NS
_CHUNK = 128
_NB = 8
_PF = 4
_NCH0 = 88
_NCH1 = 72
_TOT_CH = (_NCH0 + _NCH1) * _NS
_E_PAD = _TOT_CH * _CHUNK
_N_PAD = 10240
_RPT = _N_PAD // _NS
_NR = _N * _DIM // 128
_NRP = _N_PAD * _DIM // 128


@functools.cache
def _make_sc_segsum():
    mesh = plsc.VectorSubcoreMesh(
        core_axis_name="c", subcore_axis_name="s",
        num_cores=_NC, num_subcores=_NS)

    @functools.partial(
        pl.kernel,
        out_type=jax.ShapeDtypeStruct((_NC, _N_PAD, _DIM), jnp.float32),
        mesh=mesh,
        scratch_types=[
            pltpu.VMEM((_NCH0, _CHUNK), jnp.int32),
            pltpu.VMEM((_NCH0, _CHUNK), jnp.int32),
            [pltpu.VMEM((_CHUNK, _DIM), jnp.float32) for _ in range(_NB)],
            pltpu.VMEM_SHARED((_N, _DIM), jnp.float32),
            pltpu.VMEM_SHARED((_N_PAD, _DIM), jnp.float32),
            [pltpu.SemaphoreType.DMA for _ in range(_NB)],
            [pltpu.SemaphoreType.DMA for _ in range(_NB)],
        ],
        compiler_params=pltpu.CompilerParams(use_tc_tiling_on_sc=False),
    )
    def seg(u_hbm, srcp_hbm, dstp_hbm, zeros_hbm, out_hbm,
            src_v, dst_v, rows_v, u_sh, acc_sh, gsems, ssems):
        cid = lax.axis_index("c")
        sid = lax.axis_index("s")

        @pl.when(cid == 0)
        def _():
            pltpu.sync_copy(srcp_hbm.at[pl.ds(sid * _NCH0, _NCH0)], src_v)
            pltpu.sync_copy(dstp_hbm.at[pl.ds(sid * _NCH0, _NCH0)], dst_v)

        @pl.when(cid == 1)
        def _():
            base = _NS * _NCH0 + sid * _NCH1
            pltpu.sync_copy(srcp_hbm.at[pl.ds(base, _NCH1)],
                            src_v.at[pl.ds(0, _NCH1)])
            pltpu.sync_copy(dstp_hbm.at[pl.ds(base, _NCH1)],
                            dst_v.at[pl.ds(0, _NCH1)])

        nch = jnp.where(cid == 0, _NCH0, _NCH1)

        @pl.when(sid < _NS - 1)
        def _():
            pltpu.sync_copy(u_hbm.at[pl.ds(sid * _RPT, _RPT)],
                            u_sh.at[pl.ds(sid * _RPT, _RPT)])

        @pl.when(sid == _NS - 1)
        def _():
            pltpu.sync_copy(
                u_hbm.at[pl.ds((_NS - 1) * _RPT, _N - (_NS - 1) * _RPT)],
                u_sh.at[pl.ds((_NS - 1) * _RPT, _N - (_NS - 1) * _RPT)])

        pltpu.sync_copy(zeros_hbm.at[pl.ds(sid * _RPT, _RPT)],
                        acc_sh.at[pl.ds(sid * _RPT, _RPT)])
        plsc.subcore_barrier()

        def gather(j, b):
            pltpu.async_copy(u_sh.at[src_v.at[j]], rows_v[b], gsems[b])

        def wait_gather(j, b):
            pltpu.make_async_copy(u_sh.at[src_v.at[j]], rows_v[b],
                                  gsems[b]).wait()

        def scatter(j, b):
            pltpu.async_copy(rows_v[b], acc_sh.at[dst_v.at[j]], ssems[b],
                             add=True)

        def wait_scatter(j, b):
            pltpu.make_async_copy(rows_v[b], acc_sh.at[dst_v.at[j]],
                                  ssems[b]).wait()

        for b in range(_PF):
            gather(b, b)

        def body(g, carry):
            for b in range(_NB):
                j = g * _NB + b
                wait_gather(j, b)
                scatter(j, b)
                pre = j + _PF
                pb = (b + _PF) % _NB

                @pl.when(jnp.logical_and(pre < nch, pre >= _NB))
                def _():
                    wait_scatter(pre - _NB, pb)

                @pl.when(pre < nch)
                def _():
                    gather(pre, pb)
            return carry

        lax.fori_loop(0, nch // _NB, body, 0)
        for k in range(_NB):
            wait_scatter(nch - _NB + k, k)
        plsc.subcore_barrier()
        pltpu.sync_copy(acc_sh.at[pl.ds(sid * _RPT, _RPT)],
                        out_hbm.at[cid, pl.ds(sid * _RPT, _RPT)])

    return seg


def _dot(a, b):
    return jnp.dot(a, b, preferred_element_type=jnp.float32)


def _combine4(s):
    c = (s[:, 0 * _DIM:1 * _DIM] + s[:, 1 * _DIM:2 * _DIM]
         + s[:, 2 * _DIM:3 * _DIM] + s[:, 3 * _DIM:4 * _DIM])
    return jnp.concatenate([c, c, c, c], axis=1)


def _dense_block(u, agg, b1, w2blk, b2, gamma, beta):
    z = jnp.maximum(u + agg + b1, 0.0)
    z = _dot(z, w2blk) + b2
    mu = _combine4(jnp.sum(z, axis=0, keepdims=True)) * (1.0 / _N)
    d = z - mu
    var = _combine4(jnp.sum(d * d, axis=0, keepdims=True)) * (1.0 / _N)
    z = gamma * d / jnp.sqrt(var + 1e-5) + beta
    return jnp.maximum(z, 0.0)


def _tc_proj(x, w):
    def body(x_ref, w_ref, o_ref):
        o_ref[...] = _dot(x_ref[...], w_ref[...])

    return pl.pallas_call(
        body, out_shape=jax.ShapeDtypeStruct((_N, _DIM), jnp.float32))(x, w)


def _tc_layer(u, aggp, b1, w2blk, b2, gamma, beta, w1nblk):
    def body(u_ref, agg_ref, b1_ref, w2_ref, b2_ref, g_ref, be_ref,
             w1n_ref, o_ref):
        agg = agg_ref[0, :_NR, :] + agg_ref[1, :_NR, :]
        h = _dense_block(u_ref[...], agg, b1_ref[...], w2_ref[...],
                         b2_ref[...], g_ref[...], be_ref[...])
        o_ref[...] = _dot(h, w1n_ref[...])

    return pl.pallas_call(
        body, out_shape=jax.ShapeDtypeStruct((_NR, 128), jnp.float32))(
            u, aggp, b1, w2blk, b2, gamma, beta, w1nblk)


def _tc_final(u, aggp, b1, w2blk, b2, gamma, beta, gidt, wfc, bfc):
    def body(u_ref, agg_ref, b1_ref, w2_ref, b2_ref, g_ref, be_ref,
             gid_ref, wfc_ref, bfc_ref, o_ref):
        agg = agg_ref[0, :_NR, :] + agg_ref[1, :_NR, :]
        h = _dense_block(u_ref[...], agg, b1_ref[...], w2_ref[...],
                         b2_ref[...], g_ref[...], be_ref[...])
        rows = lax.broadcasted_iota(jnp.int32, (_G, _NR), 0)
        pooled = jnp.zeros((_G, _DIM), jnp.float32)
        for k in range(4):
            onek = (jnp.broadcast_to(gid_ref[k:k + 1, :], (_G, _NR))
                    == rows).astype(jnp.float32)
            pooled = pooled + _dot(onek, h[:, _DIM * k:_DIM * (k + 1)])
        o_ref[...] = jnp.maximum(_dot(pooled, wfc_ref[...]) + bfc_ref[...],
                                 0.0)

    return pl.pallas_call(
        body, out_shape=jax.ShapeDtypeStruct((_G, _OUT), jnp.float32))(
            u, aggp, b1, w2blk, b2, gamma, beta, gidt, wfc, bfc)


def _blockdiag(w):
    return jnp.kron(jnp.eye(4, dtype=jnp.float32), w)


def _tile4(v):
    return jnp.tile(v.reshape(1, _DIM), (1, 4))


def kernel(x, edge_index, graph_id, params):
    eidx = edge_index.reshape(2, _E // _CHUNK, _CHUNK)
    npad = _TOT_CH - _E // _CHUNK
    srcp = jnp.concatenate(
        [eidx[0], jnp.zeros((npad, _CHUNK), jnp.int32)], axis=0)
    dstp = jnp.concatenate(
        [eidx[1], jnp.full((npad, _CHUNK), _N_PAD - 1, jnp.int32)], axis=0)
    zeros = jnp.zeros((_N_PAD, _DIM), jnp.float32)
    gidt = graph_id.reshape(_NR, 4).T

    sc_segsum = _make_sc_segsum()
    u = _tc_proj(x, params["layer1"]["W1"]).reshape(_NR, 128)
    out = None
    for i in range(1, 6):
        p = params[f"layer{i}"]
        aggp = sc_segsum(u.reshape(_N, _DIM), srcp, dstp, zeros)
        agg128 = aggp.reshape(_NC, _NRP, 128)
        b1 = _tile4(p["b1"])
        b2 = _tile4(p["b2"])
        gamma = _tile4(p["gamma"])
        beta = _tile4(p["beta"])
        w2blk = _blockdiag(p["W2"])
        if i < 5:
            w1nblk = _blockdiag(params[f"layer{i + 1}"]["W1"])
            u = _tc_layer(u, agg128, b1, w2blk, b2, gamma, beta, w1nblk)
        else:
            out = _tc_final(u, agg128, b1, w2blk, b2, gamma, beta, gidt,
                            params["fc"]["W"], params["fc"]["b"].reshape(1, _OUT))
    return out
